# Initial kernel scaffold; baseline (speedup 1.0000x reference)
import jax
import jax.numpy as jnp
from jax.experimental import pallas as pl


def kernel(x, conv_weight, conv_bias):
    raise NotImplementedError("write your pallas kernel here")


# speedup vs baseline: 1.6477x; 1.6477x over previous
import functools

import jax
import jax.numpy as jnp
from jax import lax
from jax.experimental import pallas as pl
from jax.experimental.pallas import tpu as pltpu
from jax.experimental.pallas import tpu_sc as plsc

_NUM_CLASSES = 10
_LANES = 16


@functools.cache
def _build_call(batch: int):
    info = plsc.get_sparse_core_info()
    num_workers = info.num_cores * info.num_subcores
    b_per_w = batch // num_workers
    assert b_per_w * num_workers == batch and b_per_w % _LANES == 0
    flat_per_w = b_per_w * _NUM_CLASSES
    mesh = plsc.VectorSubcoreMesh(core_axis_name="c", subcore_axis_name="s")

    @functools.partial(
        pl.kernel,
        out_type=jax.ShapeDtypeStruct((batch * _NUM_CLASSES,), jnp.float32),
        mesh=mesh,
        scratch_types=[
            pltpu.VMEM((b_per_w,), jnp.int32),
            pltpu.VMEM((flat_per_w,), jnp.float32),
        ],
        compiler_params=pltpu.CompilerParams(needs_layout_passes=False),
    )
    def onehot_sc(x_hbm, out_hbm, xv, ov):
        wid = lax.axis_index("s") * info.num_cores + lax.axis_index("c")
        base = wid * b_per_w
        pltpu.sync_copy(x_hbm.at[pl.ds(base, b_per_w)], xv)

        zeros = jnp.zeros((_LANES,), jnp.float32)

        def zero_body(i, c):
            ov[pl.ds(i * _LANES, _LANES)] = zeros
            return c

        lax.fori_loop(0, flat_per_w // _LANES, zero_body, 0)

        lanes = lax.iota(jnp.int32, _LANES)
        tens = jnp.full((_LANES,), 10.0, jnp.float32)

        def scatter_body(g, c):
            vals = xv[pl.ds(g * _LANES, _LANES)]
            pos = (g * _LANES + lanes) * _NUM_CLASSES + vals
            plsc.store_scatter(ov, [pos], tens)
            return c

        lax.fori_loop(0, b_per_w // _LANES, scatter_body, 0)

        pltpu.sync_copy(ov, out_hbm.at[pl.ds(wid * flat_per_w, flat_per_w)])

    return onehot_sc


def kernel(x, conv_weight, conv_bias):
    batch = x.shape[0]
    out_flat = _build_call(batch)(x.astype(jnp.int32))
    return out_flat.reshape(batch, _NUM_CLASSES)
